# Initial kernel scaffold; baseline (speedup 1.0000x reference)
import jax
import jax.numpy as jnp
from jax.experimental import pallas as pl


def kernel(x, fitness):
    raise NotImplementedError("write your pallas kernel here")


# speedup vs baseline: 1.1636x; 1.1636x over previous
import functools

import jax
import jax.numpy as jnp
from jax import lax
from jax.experimental import pallas as pl
from jax.experimental.pallas import tpu as pltpu
from jax.experimental.pallas import tpu_sc as plsc

L = 16
N = 8192
NV = N // L
B = 32
D = 128
W = 128
NWIN = N // W
NC = 2


def _log2(v):
    return v.bit_length() - 1


@functools.partial(
    pl.kernel,
    out_type=(
        jax.ShapeDtypeStruct((B * N, D), jnp.float32),
        jax.ShapeDtypeStruct((B, N), jnp.float32),
    ),
    mesh=plsc.VectorSubcoreMesh(core_axis_name="c", subcore_axis_name="s"),
    compiler_params=pltpu.CompilerParams(needs_layout_passes=False),
    scratch_types=[
        pltpu.VMEM((N,), jnp.int32),
        pltpu.VMEM((N,), jnp.int32),
        pltpu.VMEM((N,), jnp.float32),
        pltpu.VMEM((2, W, D), jnp.float32),
        pltpu.SemaphoreType.DMA,
        pltpu.SemaphoreType.DMA,
    ],
)
def _sc_sort_gather(x_hbm, fit_hbm, y_hbm, fs_hbm,
                    key_v, idx_v, fit_v, buf_v, sem0, sem1):
    b = lax.axis_index("s") * NC + lax.axis_index("c")

    pltpu.sync_copy(fit_hbm.at[b], fit_v)

    def init_body(v, _):
        f = fit_v[pl.ds(v * L, L)]
        k = lax.bitcast_convert_type(f, jnp.int32)
        m = lax.shift_right_arithmetic(k, 31) & jnp.int32(0x7FFFFFFF)
        key_v[pl.ds(v * L, L)] = k ^ m
        idx_v[pl.ds(v * L, L)] = lax.iota(jnp.int32, L) + v * L
        return 0
    lax.fori_loop(0, NV, init_body, 0, unroll=False)

    def vreg_sort(v, descending):
        s = pl.ds(v * L, L)
        sk, sv = plsc.sort_key_val(key_v[s], idx_v[s], descending=descending)
        key_v[s] = sk
        idx_v[s] = sv

    def p0_body(t, _):
        vreg_sort(2 * t, False)
        vreg_sort(2 * t + 1, True)
        return 0
    lax.fori_loop(0, NV // 2, p0_body, 0, unroll=False)

    for K in [2 << i for i in range(_log2(NV))]:
        kb = _log2(K)
        J = K // 2
        while J >= 1:
            jb = _log2(J)

            def cx_body(t, _, J=J, jb=jb, K=K):
                v = ((t >> jb) << (jb + 1)) | (t & (J - 1))
                asc = (v & K) == 0
                sa = pl.ds(v * L, L)
                sb = pl.ds((v + J) * L, L)
                ka = key_v[sa]
                kb_ = key_v[sb]
                ia = idx_v[sa]
                ib = idx_v[sb]
                m = (ka <= kb_) == asc
                key_v[sa] = jnp.where(m, ka, kb_)
                key_v[sb] = jnp.where(m, kb_, ka)
                idx_v[sa] = jnp.where(m, ia, ib)
                idx_v[sb] = jnp.where(m, ib, ia)
                return 0
            lax.fori_loop(0, NV // 2, cx_body, 0, unroll=False)
            J //= 2

        if K == NV:
            def fs_body(t, _):
                vreg_sort(t, False)
                return 0
            lax.fori_loop(0, NV, fs_body, 0, unroll=False)
        else:
            def dir_body(t, _, kb=kb, K=K):
                v = ((t >> kb) << (kb + 1)) | (t & (K - 1))
                vreg_sort(v, False)
                vreg_sort(v | K, True)
                return 0
            lax.fori_loop(0, NV // 2, dir_body, 0, unroll=False)

    for p in (0, 1, 0, 1):
        def clean_body(t, _, p=p):
            lo = t * 2 * L + p + 2 * lax.iota(jnp.int32, L)
            hi = jnp.minimum(lo + 1, N - 1)
            klo = plsc.load_gather(key_v, [lo])
            khi = plsc.load_gather(key_v, [hi])
            ilo = plsc.load_gather(idx_v, [lo])
            ihi = plsc.load_gather(idx_v, [hi])
            sw = (klo == khi) & (ilo > ihi)
            plsc.store_scatter(idx_v, [lo], jnp.where(sw, ihi, ilo))
            plsc.store_scatter(idx_v, [hi], jnp.where(sw, ilo, ihi))
            return 0
        lax.fori_loop(0, NV // 2, clean_body, 0, unroll=False)

    def post_body(v, _):
        s = pl.ds(v * L, L)
        k = key_v[s]
        m = lax.shift_right_arithmetic(k, 31) & jnp.int32(0x7FFFFFFF)
        fit_v[s] = lax.bitcast_convert_type(k ^ m, jnp.float32)
        idx_v[s] = idx_v[s] + b * N
        return 0
    lax.fori_loop(0, NV, post_body, 0, unroll=False)

    pltpu.sync_copy(fit_v, fs_hbm.at[b])

    def gather_body(w, _):
        idx_slice = idx_v.at[pl.ds(w * W, W)]
        pltpu.async_copy(x_hbm.at[idx_slice], buf_v.at[0], sem0).wait()
        pltpu.sync_copy(buf_v.at[0], y_hbm.at[pl.ds(b * N + w * W, W)])
        return 0
    lax.fori_loop(0, NWIN, gather_body, 0, unroll=False)


def kernel(x, fitness):
    xflat = x.reshape(B * N, D)
    yflat, fit_sorted = _sc_sort_gather(xflat, fitness)
    return yflat.reshape(B, N, D), fit_sorted


# speedup vs baseline: 1.5477x; 1.3301x over previous
import functools

import jax
import jax.numpy as jnp
from jax import lax
from jax.experimental import pallas as pl
from jax.experimental.pallas import tpu as pltpu
from jax.experimental.pallas import tpu_sc as plsc

L = 16
N = 8192
NV = N // L
B = 32
D = 128
W = 128
NWIN = N // W
NC = 2

RADIX_PASSES = ((0, 2048), (11, 2048), (22, 1024))
NBINS = 2048


@functools.partial(
    pl.kernel,
    out_type=(
        jax.ShapeDtypeStruct((B * N, D), jnp.float32),
        jax.ShapeDtypeStruct((B, N), jnp.float32),
    ),
    mesh=plsc.VectorSubcoreMesh(core_axis_name="c", subcore_axis_name="s"),
    compiler_params=pltpu.CompilerParams(needs_layout_passes=False),
    scratch_types=[
        pltpu.VMEM((N,), jnp.int32),
        pltpu.VMEM((N,), jnp.int32),
        pltpu.VMEM((N,), jnp.int32),
        pltpu.VMEM((N,), jnp.int32),
        pltpu.VMEM((NBINS,), jnp.int32),
        pltpu.VMEM((N,), jnp.float32),
        pltpu.VMEM((2, W, D), jnp.float32),
        pltpu.SemaphoreType.DMA,
        pltpu.SemaphoreType.DMA,
    ],
)
def _sc_sort_gather(x_hbm, fit_hbm, y_hbm, fs_hbm,
                    key_v, idx_v, key2_v, idx2_v, hist_v, fit_v, buf_v,
                    sem0, sem1):
    b = lax.axis_index("s") * NC + lax.axis_index("c")

    pltpu.sync_copy(fit_hbm.at[b], fit_v)

    def init_body(v, _):
        f = fit_v[pl.ds(v * L, L)]
        k = lax.bitcast_convert_type(f, jnp.int32)
        m = lax.shift_right_arithmetic(k, 31)
        key_v[pl.ds(v * L, L)] = k ^ (m | jnp.int32(-0x80000000))
        idx_v[pl.ds(v * L, L)] = lax.iota(jnp.int32, L) + v * L
        return 0
    lax.fori_loop(0, NV, init_body, 0, unroll=False)

    bufs = ((key_v, idx_v, key2_v, idx2_v),
            (key2_v, idx2_v, key_v, idx_v),
            (key_v, idx_v, key2_v, idx2_v))
    for (shift, bins), (sk, si, dk, di) in zip(RADIX_PASSES, bufs):
        nhv = bins // L
        mask = jnp.int32(bins - 1)

        def zero_body(g, _):
            hist_v[pl.ds(g * L, L)] = jnp.zeros((L,), jnp.int32)
            return 0
        lax.fori_loop(0, nhv, zero_body, 0, unroll=False)

        def count_body(v, _, sk=sk, shift=shift, mask=mask):
            k = sk[pl.ds(v * L, L)]
            d = lax.shift_right_logical(k, shift) & mask
            cnt, lm = plsc.scan_count(d)
            plsc.addupdate_scatter(hist_v, [d], cnt, mask=lm)
            return 0
        lax.fori_loop(0, NV, count_body, 0, unroll=False)

        def scan_body(g, carry):
            hv = hist_v[pl.ds(g * L, L)]
            inc = plsc.cumsum(hv) + carry
            hist_v[pl.ds(g * L, L)] = inc - hv
            return carry + jnp.sum(hv)
        lax.fori_loop(0, nhv, scan_body, jnp.int32(0), unroll=False)

        def perm_body(v, _, sk=sk, si=si, dk=dk, di=di, shift=shift, mask=mask):
            k = sk[pl.ds(v * L, L)]
            i = si[pl.ds(v * L, L)]
            d = lax.shift_right_logical(k, shift) & mask
            cnt, lm = plsc.scan_count(d)
            off = plsc.load_gather(hist_v, [d])
            pos = off + cnt - 1
            plsc.store_scatter(dk, [pos], k)
            plsc.store_scatter(di, [pos], i)
            plsc.addupdate_scatter(hist_v, [d], cnt, mask=lm)
            return 0
        lax.fori_loop(0, NV, perm_body, 0, unroll=False)

    def post_body(v, _):
        s = pl.ds(v * L, L)
        u = key2_v[s]
        m2 = lax.shift_right_arithmetic(u, 31)
        k = u ^ (jnp.int32(-0x80000000) | (~m2 & jnp.int32(0x7FFFFFFF)))
        fit_v[s] = lax.bitcast_convert_type(k, jnp.float32)
        idx2_v[s] = idx2_v[s] + b * N
        return 0
    lax.fori_loop(0, NV, post_body, 0, unroll=False)

    pltpu.sync_copy(fit_v, fs_hbm.at[b])

    def gather_body(w, _):
        idx_slice = idx2_v.at[pl.ds(w * W, W)]
        pltpu.async_copy(x_hbm.at[idx_slice], buf_v.at[0], sem0).wait()
        pltpu.sync_copy(buf_v.at[0], y_hbm.at[pl.ds(b * N + w * W, W)])
        return 0
    lax.fori_loop(0, NWIN, gather_body, 0, unroll=False)


def kernel(x, fitness):
    xflat = x.reshape(B * N, D)
    yflat, fit_sorted = _sc_sort_gather(xflat, fitness)
    return yflat.reshape(B, N, D), fit_sorted


# speedup vs baseline: 1.9514x; 1.2608x over previous
import functools

import jax
import jax.numpy as jnp
from jax import lax
from jax.experimental import pallas as pl
from jax.experimental.pallas import tpu as pltpu
from jax.experimental.pallas import tpu_sc as plsc

L = 16
N = 8192
NV = N // L
B = 32
D = 128
W = 128
NWIN = N // W
NC = 2

RADIX_PASSES = ((0, 2048), (11, 2048), (22, 1024))
NBINS = 2048


@functools.partial(
    pl.kernel,
    out_type=(
        jax.ShapeDtypeStruct((B * N, D), jnp.float32),
        jax.ShapeDtypeStruct((B, N), jnp.float32),
    ),
    mesh=plsc.VectorSubcoreMesh(core_axis_name="c", subcore_axis_name="s"),
    compiler_params=pltpu.CompilerParams(needs_layout_passes=False),
    scratch_types=[
        pltpu.VMEM((N,), jnp.int32),
        pltpu.VMEM((N,), jnp.int32),
        pltpu.VMEM((N,), jnp.int32),
        pltpu.VMEM((N,), jnp.int32),
        pltpu.VMEM((NBINS,), jnp.int32),
        pltpu.VMEM((N,), jnp.float32),
        pltpu.VMEM((2, W, D), jnp.float32),
        pltpu.SemaphoreType.DMA,
        pltpu.SemaphoreType.DMA,
    ],
)
def _sc_sort_gather(x_hbm, fit_hbm, y_hbm, fs_hbm,
                    key_v, idx_v, key2_v, idx2_v, hist_v, fit_v, buf_v,
                    sem0, sem1):
    b = lax.axis_index("s") * NC + lax.axis_index("c")

    pltpu.sync_copy(fit_hbm.at[b], fit_v)

    def init_body(v, _):
        f = fit_v[pl.ds(v * L, L)]
        k = lax.bitcast_convert_type(f, jnp.int32)
        m = lax.shift_right_arithmetic(k, 31)
        key_v[pl.ds(v * L, L)] = k ^ (m | jnp.int32(-0x80000000))
        idx_v[pl.ds(v * L, L)] = lax.iota(jnp.int32, L) + v * L
        return 0
    lax.fori_loop(0, NV, init_body, 0, unroll=False)

    bufs = ((key_v, idx_v, key2_v, idx2_v),
            (key2_v, idx2_v, key_v, idx_v),
            (key_v, idx_v, key2_v, idx2_v))
    for (shift, bins), (sk, si, dk, di) in zip(RADIX_PASSES, bufs):
        nhv = bins // L
        mask = jnp.int32(bins - 1)

        def zero_body(g, _):
            hist_v[pl.ds(g * L, L)] = jnp.zeros((L,), jnp.int32)
            return 0
        lax.fori_loop(0, nhv, zero_body, 0, unroll=False)

        def count_body(v, _, sk=sk, shift=shift, mask=mask):
            k = sk[pl.ds(v * L, L)]
            d = lax.shift_right_logical(k, shift) & mask
            cnt, lm = plsc.scan_count(d)
            plsc.addupdate_scatter(hist_v, [d], cnt, mask=lm)
            return 0
        lax.fori_loop(0, NV, count_body, 0, unroll=False)

        def scan_body(g, carry):
            hv = hist_v[pl.ds(g * L, L)]
            inc = plsc.cumsum(hv) + carry
            hist_v[pl.ds(g * L, L)] = inc - hv
            return carry + jnp.sum(hv)
        lax.fori_loop(0, nhv, scan_body, jnp.int32(0), unroll=False)

        def perm_body(v, _, sk=sk, si=si, dk=dk, di=di, shift=shift, mask=mask):
            k = sk[pl.ds(v * L, L)]
            i = si[pl.ds(v * L, L)]
            d = lax.shift_right_logical(k, shift) & mask
            cnt, lm = plsc.scan_count(d)
            off = plsc.load_gather(hist_v, [d])
            pos = off + cnt - 1
            plsc.store_scatter(dk, [pos], k)
            plsc.store_scatter(di, [pos], i)
            plsc.addupdate_scatter(hist_v, [d], cnt, mask=lm)
            return 0
        lax.fori_loop(0, NV, perm_body, 0, unroll=False)

    def post_body(v, _):
        s = pl.ds(v * L, L)
        u = key2_v[s]
        m2 = lax.shift_right_arithmetic(u, 31)
        k = u ^ (jnp.int32(-0x80000000) | (~m2 & jnp.int32(0x7FFFFFFF)))
        fit_v[s] = lax.bitcast_convert_type(k, jnp.float32)
        idx2_v[s] = idx2_v[s] + b * N
        return 0
    lax.fori_loop(0, NV, post_body, 0, unroll=False)

    pltpu.sync_copy(fit_v, fs_hbm.at[b])

    sems = (sem0, sem1)

    def g_start(w, slot):
        pltpu.async_copy(x_hbm.at[idx2_v.at[pl.ds(w * W, W)]],
                         buf_v.at[slot], sems[slot])

    def g_wait(w, slot):
        pltpu.make_async_copy(x_hbm.at[idx2_v.at[pl.ds(w * W, W)]],
                              buf_v.at[slot], sems[slot]).wait()

    def g_out(w, slot):
        pltpu.sync_copy(buf_v.at[slot], y_hbm.at[pl.ds(b * N + w * W, W)])

    g_start(0, 0)
    g_start(1, 1)

    def gather_body(t, _):
        w0 = 2 * t
        g_wait(w0, 0)
        g_out(w0, 0)
        g_start(w0 + 2, 0)
        g_wait(w0 + 1, 1)
        g_out(w0 + 1, 1)
        g_start(w0 + 3, 1)
        return 0
    lax.fori_loop(0, NWIN // 2 - 1, gather_body, 0, unroll=False)
    g_wait(NWIN - 2, 0)
    g_out(NWIN - 2, 0)
    g_wait(NWIN - 1, 1)
    g_out(NWIN - 1, 1)


def kernel(x, fitness):
    xflat = x.reshape(B * N, D)
    yflat, fit_sorted = _sc_sort_gather(xflat, fitness)
    return yflat.reshape(B, N, D), fit_sorted


# speedup vs baseline: 2.2178x; 1.1366x over previous
import functools

import jax
import jax.numpy as jnp
from jax import lax
from jax.experimental import pallas as pl
from jax.experimental.pallas import tpu as pltpu
from jax.experimental.pallas import tpu_sc as plsc

L = 16
N = 8192
NV = N // L
B = 32
D = 128
W = 128
NWIN = N // W
NC = 2

BINS0, BINS1, BINS2 = 2048, 2048, 1024
SEG0, SEG1, SEG2 = 0, 2048, 4096
NHIST = BINS0 + BINS1 + BINS2


@functools.partial(
    pl.kernel,
    out_type=(
        jax.ShapeDtypeStruct((B * N, D), jnp.float32),
        jax.ShapeDtypeStruct((B, N), jnp.float32),
    ),
    mesh=plsc.VectorSubcoreMesh(core_axis_name="c", subcore_axis_name="s"),
    compiler_params=pltpu.CompilerParams(needs_layout_passes=False),
    scratch_types=[
        pltpu.VMEM((N,), jnp.int32),
        pltpu.VMEM((N,), jnp.int32),
        pltpu.VMEM((N,), jnp.int32),
        pltpu.VMEM((N,), jnp.int32),
        pltpu.VMEM((NHIST,), jnp.int32),
        pltpu.VMEM((N,), jnp.float32),
        pltpu.VMEM((2, W, D), jnp.float32),
        pltpu.SemaphoreType.DMA,
        pltpu.SemaphoreType.DMA,
    ],
)
def _sc_sort_gather(x_hbm, fit_hbm, y_hbm, fs_hbm,
                    key_v, idx_v, key2_v, idx2_v, hist_v, fit_v, buf_v,
                    sem0, sem1):
    b = lax.axis_index("s") * NC + lax.axis_index("c")

    pltpu.sync_copy(fit_hbm.at[b], fit_v)

    def zero_body(g, _):
        hist_v[pl.ds(g * L, L)] = jnp.zeros((L,), jnp.int32)
        return 0
    lax.fori_loop(0, NHIST // L, zero_body, 0, unroll=4)

    def digits(k):
        d0 = k & jnp.int32(BINS0 - 1)
        d1 = lax.shift_right_logical(k, 11) & jnp.int32(BINS1 - 1)
        d2 = lax.shift_right_logical(k, 22)
        return d0, d1 + SEG1, d2 + SEG2

    def count_body(v, _):
        f = fit_v[pl.ds(v * L, L)]
        k = lax.bitcast_convert_type(f, jnp.int32)
        k = k ^ (lax.shift_right_arithmetic(k, 31) | jnp.int32(-0x80000000))
        key_v[pl.ds(v * L, L)] = k
        idx_v[pl.ds(v * L, L)] = lax.iota(jnp.int32, L) + v * L
        for d in digits(k):
            cnt, lm = plsc.scan_count(d)
            plsc.addupdate_scatter(hist_v, [d], cnt, mask=lm)
        return 0
    lax.fori_loop(0, NV, count_body, 0, unroll=False)

    for seg, bins in ((SEG0, BINS0), (SEG1, BINS1), (SEG2, BINS2)):
        def scan_body(g, carry, seg=seg):
            s = pl.ds(seg + g * L, L)
            hv = hist_v[s]
            inc = plsc.cumsum(hv) + carry
            hist_v[s] = inc - hv
            return carry + jnp.sum(hv)
        lax.fori_loop(0, bins // L, scan_body, jnp.int32(0), unroll=False)

    def perm1_body(v, _):
        k = key_v[pl.ds(v * L, L)]
        i = idx_v[pl.ds(v * L, L)]
        d = k & jnp.int32(BINS0 - 1)
        cnt, lm = plsc.scan_count(d)
        pos = plsc.load_gather(hist_v, [d]) + cnt - 1
        plsc.store_scatter(key2_v, [pos], k)
        plsc.store_scatter(idx2_v, [pos], i)
        plsc.addupdate_scatter(hist_v, [d], cnt, mask=lm)
        return 0
    lax.fori_loop(0, NV, perm1_body, 0, unroll=False)

    def perm2_body(v, _):
        k = key2_v[pl.ds(v * L, L)]
        i = idx2_v[pl.ds(v * L, L)]
        d = (lax.shift_right_logical(k, 11) & jnp.int32(BINS1 - 1)) + SEG1
        cnt, lm = plsc.scan_count(d)
        pos = plsc.load_gather(hist_v, [d]) + cnt - 1
        plsc.store_scatter(key_v, [pos], k)
        plsc.store_scatter(idx_v, [pos], i)
        plsc.addupdate_scatter(hist_v, [d], cnt, mask=lm)
        return 0
    lax.fori_loop(0, NV, perm2_body, 0, unroll=False)

    def perm3_body(v, _):
        k = key_v[pl.ds(v * L, L)]
        i = idx_v[pl.ds(v * L, L)]
        d = lax.shift_right_logical(k, 22) + SEG2
        cnt, lm = plsc.scan_count(d)
        pos = plsc.load_gather(hist_v, [d]) + cnt - 1
        m2 = lax.shift_right_arithmetic(k, 31)
        kf = k ^ (jnp.int32(-0x80000000) | (~m2 & jnp.int32(0x7FFFFFFF)))
        plsc.store_scatter(fit_v, [pos], lax.bitcast_convert_type(kf, jnp.float32))
        plsc.store_scatter(idx2_v, [pos], i + b * N)
        plsc.addupdate_scatter(hist_v, [d], cnt, mask=lm)
        return 0
    lax.fori_loop(0, NV, perm3_body, 0, unroll=False)

    pltpu.sync_copy(fit_v, fs_hbm.at[b])

    sems = (sem0, sem1)

    def g_start(w, slot):
        pltpu.async_copy(x_hbm.at[idx2_v.at[pl.ds(w * W, W)]],
                         buf_v.at[slot], sems[slot])

    def g_wait(w, slot):
        pltpu.make_async_copy(x_hbm.at[idx2_v.at[pl.ds(w * W, W)]],
                              buf_v.at[slot], sems[slot]).wait()

    def g_out(w, slot):
        pltpu.sync_copy(buf_v.at[slot], y_hbm.at[pl.ds(b * N + w * W, W)])

    g_start(0, 0)
    g_start(1, 1)

    def gather_body(t, _):
        w0 = 2 * t
        g_wait(w0, 0)
        g_out(w0, 0)
        g_start(w0 + 2, 0)
        g_wait(w0 + 1, 1)
        g_out(w0 + 1, 1)
        g_start(w0 + 3, 1)
        return 0
    lax.fori_loop(0, NWIN // 2 - 1, gather_body, 0, unroll=False)
    g_wait(NWIN - 2, 0)
    g_out(NWIN - 2, 0)
    g_wait(NWIN - 1, 1)
    g_out(NWIN - 1, 1)


def kernel(x, fitness):
    xflat = x.reshape(B * N, D)
    yflat, fit_sorted = _sc_sort_gather(xflat, fitness)
    return yflat.reshape(B, N, D), fit_sorted
